# baseline (speedup 1.0000x reference)
import functools

import jax
import jax.numpy as jnp
from jax import lax
from jax.experimental import pallas as pl
from jax.experimental.pallas import tpu as pltpu
from jax.experimental.pallas import tpu_sc as plsc

N_NODES = 10000
D = 128
N_EDGES = 320000

NC = 2
NS = 16
NW = NC * NS
EPT = N_EDGES // NW
CHUNK = 80
NCHUNK = EPT // CHUNK
NP = 10240
RPT = NP // NS
RZ = 128

_mesh = plsc.VectorSubcoreMesh(
    core_axis_name="c", subcore_axis_name="s", num_cores=NC, num_subcores=NS
)
_sc_params = pltpu.CompilerParams(use_tc_tiling_on_sc=False)


def _stage_chunk(dst_c, stage, i):
    for k in range(CHUNK // 16):
        off = pl.multiple_of(i * CHUNK + k * 16, 16)
        dst_c[pl.ds(k * 16, 16)] = stage[pl.ds(off, 16)]


@functools.partial(
    pl.kernel,
    mesh=_mesh,
    compiler_params=_sc_params,
    out_type=[
        jax.ShapeDtypeStruct((NP,), jnp.float32),
        jax.ShapeDtypeStruct((NP,), jnp.float32),
    ],
    scratch_types=[
        pltpu.VMEM((EPT,), jnp.int32),
        pltpu.VMEM((CHUNK,), jnp.int32),
        pltpu.VMEM((CHUNK,), jnp.float32),
        pltpu.VMEM((RPT,), jnp.float32),
        pltpu.VMEM_SHARED((NP,), jnp.float32),
    ],
)
def _sc_degree(dst_hbm, d0_hbm, d1_hbm, dst_v, dst_c, ones_v, zb_v, acc_sh):
    cid = lax.axis_index("c")
    sid = lax.axis_index("s")
    wid = cid * NS + sid
    ebase = pl.multiple_of(wid * EPT, EPT)
    row0 = pl.multiple_of(sid * RPT, RPT)

    pltpu.sync_copy(dst_hbm.at[pl.ds(ebase, EPT)], dst_v)

    def _fill(i, _):
        ones_v[pl.ds(i * 16, 16)] = jnp.ones((16,), jnp.float32)
        return 0

    lax.fori_loop(0, CHUNK // 16, _fill, 0)

    def _zero(i, _):
        zb_v[pl.ds(i * 16, 16)] = jnp.zeros((16,), jnp.float32)
        return 0

    lax.fori_loop(0, RPT // 16, _zero, 0)
    pltpu.sync_copy(zb_v, acc_sh.at[pl.ds(row0, RPT)])
    plsc.subcore_barrier()

    def _chunk(i, _):
        _stage_chunk(dst_c, dst_v, i)
        pltpu.sync_copy(ones_v, acc_sh.at[dst_c], add=True)
        return 0

    lax.fori_loop(0, NCHUNK, _chunk, 0)
    plsc.subcore_barrier()

    @pl.when(cid == 0)
    def _():
        pltpu.sync_copy(acc_sh.at[pl.ds(row0, RPT)], d0_hbm.at[pl.ds(row0, RPT)])

    @pl.when(cid == 1)
    def _():
        pltpu.sync_copy(acc_sh.at[pl.ds(row0, RPT)], d1_hbm.at[pl.ds(row0, RPT)])


@functools.partial(
    pl.kernel,
    mesh=_mesh,
    compiler_params=_sc_params,
    out_type=[
        jax.ShapeDtypeStruct((NP, D), jnp.float32),
        jax.ShapeDtypeStruct((NP, D), jnp.float32),
    ],
    scratch_types=[
        pltpu.VMEM((EPT,), jnp.int32),
        pltpu.VMEM((EPT,), jnp.int32),
        pltpu.VMEM((CHUNK,), jnp.int32),
        pltpu.VMEM((CHUNK,), jnp.int32),
        pltpu.VMEM((CHUNK, D), jnp.float32),
        pltpu.VMEM((CHUNK, D), jnp.float32),
        pltpu.VMEM_SHARED((NP, D), jnp.float32),
        pltpu.SemaphoreType.DMA,
        pltpu.SemaphoreType.DMA,
        pltpu.SemaphoreType.DMA,
        pltpu.SemaphoreType.DMA,
    ],
)
def _sc_aggregate(y_hbm, src_hbm, dst_hbm, p0_hbm, p1_hbm,
                  src_v, dst_v, dst_ca, dst_cb, rows_a, rows_b, acc_sh,
                  sga, sgb, ssa, ssb):
    cid = lax.axis_index("c")
    sid = lax.axis_index("s")
    wid = cid * NS + sid
    ebase = pl.multiple_of(wid * EPT, EPT)
    row0 = pl.multiple_of(sid * RPT, RPT)

    pltpu.sync_copy(src_hbm.at[pl.ds(ebase, EPT)], src_v)
    pltpu.sync_copy(dst_hbm.at[pl.ds(ebase, EPT)], dst_v)

    @pl.when(cid == 0)
    def _():
        pltpu.sync_copy(y_hbm.at[pl.ds(row0, RPT)], acc_sh.at[pl.ds(row0, RPT)])

    @pl.when(cid == 1)
    def _():
        def _zero(i, _):
            for j in range(D // 16):
                rows_a[i, pl.ds(j * 16, 16)] = jnp.zeros((16,), jnp.float32)
            return 0

        lax.fori_loop(0, CHUNK, _zero, 0)
        for j in range(RPT // CHUNK):
            pltpu.sync_copy(rows_a, acc_sh.at[pl.ds(row0 + j * CHUNK, CHUNK)])

    plsc.subcore_barrier()

    def _start_gather(i, rows, dst_c, sem):
        off = pl.multiple_of(i * CHUNK, CHUNK)
        _stage_chunk(dst_c, dst_v, i)
        pltpu.async_copy(y_hbm.at[src_v.at[pl.ds(off, CHUNK)]], rows, sem)

    def _wait_gather(rows, sem):
        pltpu.make_async_copy(y_hbm.at[src_v.at[pl.ds(0, CHUNK)]], rows, sem).wait()

    def _start_scatter(rows, dst_c, sem):
        pltpu.async_copy(rows, acc_sh.at[dst_c], sem, add=True)

    def _wait_scatter(rows, dst_c, sem):
        pltpu.make_async_copy(rows, acc_sh.at[dst_c], sem).wait()

    _start_gather(0, rows_a, dst_ca, sga)
    _start_gather(1, rows_b, dst_cb, sgb)

    def _pair(k, _):
        i = 2 * k
        _wait_gather(rows_a, sga)
        _start_scatter(rows_a, dst_ca, ssa)
        _wait_gather(rows_b, sgb)
        _start_scatter(rows_b, dst_cb, ssb)
        _wait_scatter(rows_a, dst_ca, ssa)
        _start_gather(i + 2, rows_a, dst_ca, sga)

        @pl.when(i + 3 < NCHUNK)
        def _():
            _wait_scatter(rows_b, dst_cb, ssb)
            _start_gather(i + 3, rows_b, dst_cb, sgb)

        return 0

    lax.fori_loop(0, (NCHUNK - 1) // 2, _pair, 0)
    _wait_scatter(rows_b, dst_cb, ssb)
    _wait_gather(rows_a, sga)
    pltpu.sync_copy(rows_a, acc_sh.at[dst_ca], add=True)
    plsc.subcore_barrier()

    @pl.when(cid == 0)
    def _():
        pltpu.sync_copy(acc_sh.at[pl.ds(row0, RPT)], p0_hbm.at[pl.ds(row0, RPT)])

    @pl.when(cid == 1)
    def _():
        pltpu.sync_copy(acc_sh.at[pl.ds(row0, RPT)], p1_hbm.at[pl.ds(row0, RPT)])


RB = 512
NG = NP // RB


def _dinv(d0, d1):
    return lax.rsqrt(d0 + d1 + 1.0)


def _y1_body(x_ref, w_ref, d0_ref, d1_ref, o_ref):
    dinv = _dinv(d0_ref[...], d1_ref[...])
    xw = jnp.dot(x_ref[...], w_ref[...], preferred_element_type=jnp.float32)
    o_ref[...] = xw * dinv


def _mid_body(p0_ref, p1_ref, d0_ref, d1_ref, b_ref, w_ref, o_ref):
    dinv = _dinv(d0_ref[...], d1_ref[...])
    h = jnp.maximum(dinv * (p0_ref[...] + p1_ref[...]) + b_ref[...], 0.0)
    o_ref[...] = dinv * jnp.dot(h, w_ref[...], preferred_element_type=jnp.float32)


def _fin_body(p0_ref, p1_ref, d0_ref, d1_ref, b_ref, o_ref):
    dinv = _dinv(d0_ref[...], d1_ref[...])
    o_ref[...] = dinv * (p0_ref[...] + p1_ref[...]) + b_ref[...]


_row_spec = pl.BlockSpec((RB, D), lambda i: (i, 0))
_deg_spec = pl.BlockSpec((RB, 1), lambda i: (i, 0))
_w_spec = pl.BlockSpec((D, D), lambda i: (0, 0))
_b_spec = pl.BlockSpec((1, D), lambda i: (0, 0))
_out_sds = jax.ShapeDtypeStruct((NP, D), jnp.float32)

_tc_y1 = pl.pallas_call(
    _y1_body,
    grid=(NG,),
    in_specs=[_row_spec, _w_spec, _deg_spec, _deg_spec],
    out_specs=_row_spec,
    out_shape=_out_sds,
)

_tc_mid = pl.pallas_call(
    _mid_body,
    grid=(NG,),
    in_specs=[_row_spec, _row_spec, _deg_spec, _deg_spec, _b_spec, _w_spec],
    out_specs=_row_spec,
    out_shape=_out_sds,
)

_tc_fin = pl.pallas_call(
    _fin_body,
    grid=(NG,),
    in_specs=[_row_spec, _row_spec, _deg_spec, _deg_spec, _b_spec],
    out_specs=_row_spec,
    out_shape=_out_sds,
)


@jax.jit
def kernel(x, edge_index, W1, b1, W2, b2):
    src = edge_index[0].astype(jnp.int32)
    dst = edge_index[1].astype(jnp.int32)

    dd0, dd1 = _sc_degree(dst)
    d0 = dd0.reshape(NP, 1)
    d1 = dd1.reshape(NP, 1)
    xp = jnp.pad(x, ((0, NP - N_NODES), (0, 0)))
    b1r = b1.reshape(1, D)
    b2r = b2.reshape(1, D)

    y1 = _tc_y1(xp, W1, d0, d1)
    p0, p1 = _sc_aggregate(y1, src, dst)
    y2 = _tc_mid(p0, p1, d0, d1, b1r, W2)
    q0, q1 = _sc_aggregate(y2, src, dst)
    out = _tc_fin(q0, q1, d0, d1, b2r)
    return out[:N_NODES]


# speedup vs baseline: 1.2118x; 1.2118x over previous
import functools

import jax
import jax.numpy as jnp
from jax import lax
from jax.experimental import pallas as pl
from jax.experimental.pallas import tpu as pltpu
from jax.experimental.pallas import tpu_sc as plsc

N_NODES = 10000
D = 128
N_EDGES = 320000

NC = 2
NS = 16
NW = NC * NS
EPT = N_EDGES // NW
CHUNK = 80
NCHUNK = EPT // CHUNK
NP = 10240
RPT = NP // NS
RZ = 128

_mesh = plsc.VectorSubcoreMesh(
    core_axis_name="c", subcore_axis_name="s", num_cores=NC, num_subcores=NS
)
_sc_params = pltpu.CompilerParams(use_tc_tiling_on_sc=False)


@functools.partial(
    pl.kernel,
    mesh=_mesh,
    compiler_params=_sc_params,
    out_type=[
        jax.ShapeDtypeStruct((NP,), jnp.float32),
        jax.ShapeDtypeStruct((NP,), jnp.float32),
    ],
    scratch_types=[
        pltpu.VMEM((EPT,), jnp.int32),
        pltpu.VMEM((CHUNK,), jnp.float32),
        pltpu.VMEM((RPT,), jnp.float32),
        pltpu.VMEM_SHARED((NP,), jnp.float32),
    ],
)
def _sc_degree(dst_hbm, d0_hbm, d1_hbm, dst_v, ones_v, zb_v, acc_sh):
    cid = lax.axis_index("c")
    sid = lax.axis_index("s")
    wid = cid * NS + sid
    ebase = pl.multiple_of(wid * EPT, EPT)
    row0 = pl.multiple_of(sid * RPT, RPT)

    pltpu.sync_copy(dst_hbm.at[pl.ds(ebase, EPT)], dst_v)

    def _fill(i, _):
        ones_v[pl.ds(i * 16, 16)] = jnp.ones((16,), jnp.float32)
        return 0

    lax.fori_loop(0, CHUNK // 16, _fill, 0)

    def _zero(i, _):
        zb_v[pl.ds(i * 16, 16)] = jnp.zeros((16,), jnp.float32)
        return 0

    lax.fori_loop(0, RPT // 16, _zero, 0)
    pltpu.sync_copy(zb_v, acc_sh.at[pl.ds(row0, RPT)])
    plsc.subcore_barrier()

    def _chunk(i, _):
        off = pl.multiple_of(i * CHUNK, CHUNK)
        pltpu.sync_copy(ones_v, acc_sh.at[dst_v.at[pl.ds(off, CHUNK)]], add=True)
        return 0

    lax.fori_loop(0, NCHUNK, _chunk, 0)
    plsc.subcore_barrier()

    @pl.when(cid == 0)
    def _():
        pltpu.sync_copy(acc_sh.at[pl.ds(row0, RPT)], d0_hbm.at[pl.ds(row0, RPT)])

    @pl.when(cid == 1)
    def _():
        pltpu.sync_copy(acc_sh.at[pl.ds(row0, RPT)], d1_hbm.at[pl.ds(row0, RPT)])


@functools.partial(
    pl.kernel,
    mesh=_mesh,
    compiler_params=_sc_params,
    out_type=[
        jax.ShapeDtypeStruct((NP, D), jnp.float32),
        jax.ShapeDtypeStruct((NP, D), jnp.float32),
    ],
    scratch_types=[
        pltpu.VMEM((EPT,), jnp.int32),
        pltpu.VMEM((EPT,), jnp.int32),
        pltpu.VMEM((CHUNK, D), jnp.float32),
        pltpu.VMEM((CHUNK, D), jnp.float32),
        pltpu.VMEM_SHARED((NP, D), jnp.float32),
        pltpu.SemaphoreType.DMA,
        pltpu.SemaphoreType.DMA,
    ],
)
def _sc_aggregate(y_hbm, src_hbm, dst_hbm, p0_hbm, p1_hbm,
                  src_v, dst_v, rows_a, rows_b, acc_sh,
                  sga, sgb):
    cid = lax.axis_index("c")
    sid = lax.axis_index("s")
    wid = cid * NS + sid
    ebase = pl.multiple_of(wid * EPT, EPT)
    row0 = pl.multiple_of(sid * RPT, RPT)

    pltpu.sync_copy(src_hbm.at[pl.ds(ebase, EPT)], src_v)
    pltpu.sync_copy(dst_hbm.at[pl.ds(ebase, EPT)], dst_v)

    @pl.when(cid == 0)
    def _():
        pltpu.sync_copy(y_hbm.at[pl.ds(row0, RPT)], acc_sh.at[pl.ds(row0, RPT)])

    @pl.when(cid == 1)
    def _():
        def _zero(i, _):
            for j in range(D // 16):
                rows_a[i, pl.ds(j * 16, 16)] = jnp.zeros((16,), jnp.float32)
            return 0

        lax.fori_loop(0, CHUNK, _zero, 0)
        for j in range(RPT // CHUNK):
            pltpu.sync_copy(rows_a, acc_sh.at[pl.ds(row0 + j * CHUNK, CHUNK)])

    plsc.subcore_barrier()

    def _start_gather(i, rows, sem):
        off = pl.multiple_of(i * CHUNK, CHUNK)
        pltpu.async_copy(y_hbm.at[src_v.at[pl.ds(off, CHUNK)]], rows, sem)

    def _wait_gather(rows, sem):
        pltpu.make_async_copy(y_hbm.at[src_v.at[pl.ds(0, CHUNK)]], rows, sem).wait()

    def _scatter(i, rows):
        off = pl.multiple_of(i * CHUNK, CHUNK)
        pltpu.sync_copy(rows, acc_sh.at[dst_v.at[pl.ds(off, CHUNK)]], add=True)

    _start_gather(0, rows_a, sga)
    _start_gather(1, rows_b, sgb)

    def _pair(k, _):
        i = 2 * k
        _wait_gather(rows_a, sga)
        _scatter(i, rows_a)
        _start_gather(i + 2, rows_a, sga)
        _wait_gather(rows_b, sgb)
        _scatter(i + 1, rows_b)

        @pl.when(i + 3 < NCHUNK)
        def _():
            _start_gather(i + 3, rows_b, sgb)

        return 0

    lax.fori_loop(0, (NCHUNK - 1) // 2, _pair, 0)
    _wait_gather(rows_a, sga)
    _scatter(NCHUNK - 1, rows_a)
    plsc.subcore_barrier()

    @pl.when(cid == 0)
    def _():
        pltpu.sync_copy(acc_sh.at[pl.ds(row0, RPT)], p0_hbm.at[pl.ds(row0, RPT)])

    @pl.when(cid == 1)
    def _():
        pltpu.sync_copy(acc_sh.at[pl.ds(row0, RPT)], p1_hbm.at[pl.ds(row0, RPT)])


RB = 400
NG = N_NODES // RB


def _dinv(d0, d1):
    return lax.rsqrt(d0 + d1 + 1.0)


def _y1_body(x_ref, w_ref, d0_ref, d1_ref, o_ref):
    dinv = _dinv(d0_ref[...], d1_ref[...])
    xw = jnp.dot(x_ref[...], w_ref[...], preferred_element_type=jnp.float32)
    o_ref[...] = xw * dinv


def _mid_body(p0_ref, p1_ref, d0_ref, d1_ref, b_ref, w_ref, o_ref):
    dinv = _dinv(d0_ref[...], d1_ref[...])
    h = jnp.maximum(dinv * (p0_ref[...] + p1_ref[...]) + b_ref[...], 0.0)
    o_ref[...] = dinv * jnp.dot(h, w_ref[...], preferred_element_type=jnp.float32)


def _fin_body(p0_ref, p1_ref, d0_ref, d1_ref, b_ref, o_ref):
    dinv = _dinv(d0_ref[...], d1_ref[...])
    o_ref[...] = dinv * (p0_ref[...] + p1_ref[...]) + b_ref[...]


_row_spec = pl.BlockSpec((RB, D), lambda i: (i, 0))
_deg_spec = pl.BlockSpec((RB, 1), lambda i: (i, 0))
_w_spec = pl.BlockSpec((D, D), lambda i: (0, 0))
_b_spec = pl.BlockSpec((1, D), lambda i: (0, 0))
_out_sds = jax.ShapeDtypeStruct((NP, D), jnp.float32)

_tc_y1 = pl.pallas_call(
    _y1_body,
    grid=(NG,),
    in_specs=[_row_spec, _w_spec, _deg_spec, _deg_spec],
    out_specs=_row_spec,
    out_shape=_out_sds,
)

_tc_mid = pl.pallas_call(
    _mid_body,
    grid=(NG,),
    in_specs=[_row_spec, _row_spec, _deg_spec, _deg_spec, _b_spec, _w_spec],
    out_specs=_row_spec,
    out_shape=_out_sds,
)

_tc_fin = pl.pallas_call(
    _fin_body,
    grid=(NG,),
    in_specs=[_row_spec, _row_spec, _deg_spec, _deg_spec, _b_spec],
    out_specs=_row_spec,
    out_shape=jax.ShapeDtypeStruct((N_NODES, D), jnp.float32),
)


@jax.jit
def kernel(x, edge_index, W1, b1, W2, b2):
    src = edge_index[0].astype(jnp.int32)
    dst = edge_index[1].astype(jnp.int32)

    dd0, dd1 = _sc_degree(dst)
    d0 = dd0.reshape(NP, 1)
    d1 = dd1.reshape(NP, 1)
    b1r = b1.reshape(1, D)
    b2r = b2.reshape(1, D)

    y1 = _tc_y1(x, W1, d0, d1)
    p0, p1 = _sc_aggregate(y1, src, dst)
    y2 = _tc_mid(p0, p1, d0, d1, b1r, W2)
    q0, q1 = _sc_aggregate(y2, src, dst)
    return _tc_fin(q0, q1, d0, d1, b2r)


# speedup vs baseline: 1.2585x; 1.0386x over previous
import functools

import jax
import jax.numpy as jnp
from jax import lax
from jax.experimental import pallas as pl
from jax.experimental.pallas import tpu as pltpu
from jax.experimental.pallas import tpu_sc as plsc

N_NODES = 10000
D = 128
N_EDGES = 320000

NC = 2
NS = 16
NW = NC * NS
EPT = N_EDGES // NW
CHUNK = 80
NCHUNK = EPT // CHUNK
C2 = 128
NCH2 = 78
EPT2 = NCH2 * C2
HALF = NCH2 // 2
NREG = NW * EPT2
NP = 10240
RPT = NP // NS
RZ = 128

_mesh = plsc.VectorSubcoreMesh(
    core_axis_name="c", subcore_axis_name="s", num_cores=NC, num_subcores=NS
)
_sc_params = pltpu.CompilerParams(use_tc_tiling_on_sc=False)


@functools.partial(
    pl.kernel,
    mesh=_mesh,
    compiler_params=_sc_params,
    out_type=[
        jax.ShapeDtypeStruct((NP,), jnp.float32),
        jax.ShapeDtypeStruct((NP,), jnp.float32),
    ],
    scratch_types=[
        pltpu.VMEM((EPT,), jnp.int32),
        pltpu.VMEM((CHUNK,), jnp.float32),
        pltpu.VMEM((RPT,), jnp.float32),
        pltpu.VMEM_SHARED((NP,), jnp.float32),
    ],
)
def _sc_degree(dst_hbm, d0_hbm, d1_hbm, dst_v, ones_v, zb_v, acc_sh):
    cid = lax.axis_index("c")
    sid = lax.axis_index("s")
    wid = cid * NS + sid
    ebase = pl.multiple_of(wid * EPT, EPT)
    row0 = pl.multiple_of(sid * RPT, RPT)

    pltpu.sync_copy(dst_hbm.at[pl.ds(ebase, EPT)], dst_v)

    def _fill(i, _):
        ones_v[pl.ds(i * 16, 16)] = jnp.ones((16,), jnp.float32)
        return 0

    lax.fori_loop(0, CHUNK // 16, _fill, 0)

    def _zero(i, _):
        zb_v[pl.ds(i * 16, 16)] = jnp.zeros((16,), jnp.float32)
        return 0

    lax.fori_loop(0, RPT // 16, _zero, 0)
    pltpu.sync_copy(zb_v, acc_sh.at[pl.ds(row0, RPT)])
    plsc.subcore_barrier()

    def _chunk(i, _):
        off = pl.multiple_of(i * CHUNK, CHUNK)
        pltpu.sync_copy(ones_v, acc_sh.at[dst_v.at[pl.ds(off, CHUNK)]], add=True)
        return 0

    lax.fori_loop(0, NCHUNK, _chunk, 0)
    plsc.subcore_barrier()

    @pl.when(cid == 0)
    def _():
        pltpu.sync_copy(acc_sh.at[pl.ds(row0, RPT)], d0_hbm.at[pl.ds(row0, RPT)])

    @pl.when(cid == 1)
    def _():
        pltpu.sync_copy(acc_sh.at[pl.ds(row0, RPT)], d1_hbm.at[pl.ds(row0, RPT)])


@functools.partial(
    pl.kernel,
    mesh=_mesh,
    compiler_params=_sc_params,
    out_type=[
        jax.ShapeDtypeStruct((NP, D), jnp.float32),
        jax.ShapeDtypeStruct((NP, D), jnp.float32),
    ],
    scratch_types=[
        pltpu.VMEM((HALF * C2,), jnp.int32),
        pltpu.VMEM((NCH2 * C2,), jnp.int32),
        pltpu.VMEM((C2,), jnp.int32),
        pltpu.VMEM((C2,), jnp.int32),
        pltpu.VMEM((C2, D), jnp.float32),
        pltpu.VMEM((C2, D), jnp.float32),
        pltpu.VMEM_SHARED((NP, D), jnp.float32),
        pltpu.SemaphoreType.DMA,
        pltpu.SemaphoreType.DMA,
    ],
)
def _sc_aggregate(y_hbm, src_hbm, dst_hbm, p0_hbm, p1_hbm,
                  src_v, dst_v, src_c, dst_c, rows_a, rows_b, acc_sh,
                  sga, sgb):
    cid = lax.axis_index("c")
    sid = lax.axis_index("s")
    wid = cid * NS + sid
    ebase = pl.multiple_of(wid * EPT2, 8)
    row0 = pl.multiple_of(sid * RPT, RPT)

    pltpu.sync_copy(dst_hbm.at[pl.ds(ebase, NCH2 * C2)], dst_v)

    @pl.when(cid == 0)
    def _():
        pltpu.sync_copy(y_hbm.at[pl.ds(row0, RPT)], acc_sh.at[pl.ds(row0, RPT)])

    @pl.when(cid == 1)
    def _():
        def _zero(i, _):
            for j in range(D // 16):
                rows_a[i, pl.ds(j * 16, 16)] = jnp.zeros((16,), jnp.float32)
            return 0

        lax.fori_loop(0, C2, _zero, 0)
        for j in range(RPT // C2):
            pltpu.sync_copy(rows_a, acc_sh.at[pl.ds(row0 + j * C2, C2)])

    plsc.subcore_barrier()

    @pl.when(wid < 4)
    def _():
        lbase = pl.multiple_of(NREG + wid * C2, 8)
        pltpu.sync_copy(src_hbm.at[pl.ds(lbase, C2)], src_c)
        pltpu.sync_copy(dst_hbm.at[pl.ds(lbase, C2)], dst_c)
        pltpu.async_copy(y_hbm.at[src_c], rows_a, sga).wait()
        pltpu.sync_copy(rows_a, acc_sh.at[dst_c], add=True)

    def _start_gather(j, rows, sem):
        off = pl.multiple_of(j * C2, C2)
        pltpu.async_copy(y_hbm.at[src_v.at[pl.ds(off, C2)]], rows, sem)

    def _wait_gather(rows, sem):
        pltpu.make_async_copy(y_hbm.at[src_v.at[pl.ds(0, C2)]], rows, sem).wait()

    def _scatter(i, rows):
        off = pl.multiple_of(i * C2, C2)
        pltpu.sync_copy(rows, acc_sh.at[dst_v.at[pl.ds(off, C2)]], add=True)

    for p in range(2):
        pltpu.sync_copy(
            src_hbm.at[pl.ds(pl.multiple_of(ebase + p * HALF * C2, 8), HALF * C2)],
            src_v,
        )
        pbase = p * HALF

        _start_gather(0, rows_a, sga)
        _start_gather(1, rows_b, sgb)

        def _pair(k, _):
            j = 2 * k
            _wait_gather(rows_a, sga)
            _scatter(pbase + j, rows_a)
            _start_gather(j + 2, rows_a, sga)
            _wait_gather(rows_b, sgb)
            _scatter(pbase + j + 1, rows_b)

            @pl.when(j + 3 < HALF)
            def _():
                _start_gather(j + 3, rows_b, sgb)

            return 0

        lax.fori_loop(0, (HALF - 1) // 2, _pair, 0)
        _wait_gather(rows_a, sga)
        _scatter(pbase + HALF - 1, rows_a)

    plsc.subcore_barrier()

    @pl.when(cid == 0)
    def _():
        pltpu.sync_copy(acc_sh.at[pl.ds(row0, RPT)], p0_hbm.at[pl.ds(row0, RPT)])

    @pl.when(cid == 1)
    def _():
        pltpu.sync_copy(acc_sh.at[pl.ds(row0, RPT)], p1_hbm.at[pl.ds(row0, RPT)])


RB = 400
NG = N_NODES // RB


def _dinv(d0, d1):
    return lax.rsqrt(d0 + d1 + 1.0)


def _y1_body(x_ref, w_ref, d0_ref, d1_ref, o_ref):
    dinv = _dinv(d0_ref[...], d1_ref[...])
    xw = jnp.dot(x_ref[...], w_ref[...], preferred_element_type=jnp.float32)
    o_ref[...] = xw * dinv


def _mid_body(p0_ref, p1_ref, d0_ref, d1_ref, b_ref, w_ref, o_ref):
    dinv = _dinv(d0_ref[...], d1_ref[...])
    h = jnp.maximum(dinv * (p0_ref[...] + p1_ref[...]) + b_ref[...], 0.0)
    o_ref[...] = dinv * jnp.dot(h, w_ref[...], preferred_element_type=jnp.float32)


def _fin_body(p0_ref, p1_ref, d0_ref, d1_ref, b_ref, o_ref):
    dinv = _dinv(d0_ref[...], d1_ref[...])
    o_ref[...] = dinv * (p0_ref[...] + p1_ref[...]) + b_ref[...]


_row_spec = pl.BlockSpec((RB, D), lambda i: (i, 0))
_deg_spec = pl.BlockSpec((RB, 1), lambda i: (i, 0))
_w_spec = pl.BlockSpec((D, D), lambda i: (0, 0))
_b_spec = pl.BlockSpec((1, D), lambda i: (0, 0))
_out_sds = jax.ShapeDtypeStruct((NP, D), jnp.float32)

_tc_y1 = pl.pallas_call(
    _y1_body,
    grid=(NG,),
    in_specs=[_row_spec, _w_spec, _deg_spec, _deg_spec],
    out_specs=_row_spec,
    out_shape=_out_sds,
)

_tc_mid = pl.pallas_call(
    _mid_body,
    grid=(NG,),
    in_specs=[_row_spec, _row_spec, _deg_spec, _deg_spec, _b_spec, _w_spec],
    out_specs=_row_spec,
    out_shape=_out_sds,
)

_tc_fin = pl.pallas_call(
    _fin_body,
    grid=(NG,),
    in_specs=[_row_spec, _row_spec, _deg_spec, _deg_spec, _b_spec],
    out_specs=_row_spec,
    out_shape=jax.ShapeDtypeStruct((N_NODES, D), jnp.float32),
)


@jax.jit
def kernel(x, edge_index, W1, b1, W2, b2):
    src = edge_index[0].astype(jnp.int32)
    dst = edge_index[1].astype(jnp.int32)

    dd0, dd1 = _sc_degree(dst)
    d0 = dd0.reshape(NP, 1)
    d1 = dd1.reshape(NP, 1)
    b1r = b1.reshape(1, D)
    b2r = b2.reshape(1, D)

    y1 = _tc_y1(x, W1, d0, d1)
    p0, p1 = _sc_aggregate(y1, src, dst)
    y2 = _tc_mid(p0, p1, d0, d1, b1r, W2)
    q0, q1 = _sc_aggregate(y2, src, dst)
    return _tc_fin(q0, q1, d0, d1, b2r)


# speedup vs baseline: 1.2595x; 1.0008x over previous
import functools

import jax
import jax.numpy as jnp
from jax import lax
from jax.experimental import pallas as pl
from jax.experimental.pallas import tpu as pltpu
from jax.experimental.pallas import tpu_sc as plsc

N_NODES = 10000
D = 128
N_EDGES = 320000

NC = 2
NS = 16
NW = NC * NS
EPT = N_EDGES // NW
CHUNK = 80
NCHUNK = EPT // CHUNK
C2 = 128
NCH2 = 78
EPT2 = NCH2 * C2
HALF = NCH2 // 2
NREG = NW * EPT2
NP = 10240
RPT = NP // NS

_mesh = plsc.VectorSubcoreMesh(
    core_axis_name="c", subcore_axis_name="s", num_cores=NC, num_subcores=NS
)
_sc_params = pltpu.CompilerParams(use_tc_tiling_on_sc=False)


@functools.partial(
    pl.kernel,
    mesh=_mesh,
    compiler_params=_sc_params,
    out_type=[
        jax.ShapeDtypeStruct((NP,), jnp.float32),
        jax.ShapeDtypeStruct((NP,), jnp.float32),
    ],
    scratch_types=[
        pltpu.VMEM((EPT,), jnp.int32),
        pltpu.VMEM((CHUNK,), jnp.float32),
        pltpu.VMEM((RPT,), jnp.float32),
        pltpu.VMEM_SHARED((NP,), jnp.float32),
    ],
)
def _sc_degree(dst_hbm, d0_hbm, d1_hbm, dst_v, ones_v, zb_v, acc_sh):
    cid = lax.axis_index("c")
    sid = lax.axis_index("s")
    wid = cid * NS + sid
    ebase = pl.multiple_of(wid * EPT, EPT)
    row0 = pl.multiple_of(sid * RPT, RPT)

    pltpu.sync_copy(dst_hbm.at[pl.ds(ebase, EPT)], dst_v)

    def _fill(i, _):
        ones_v[pl.ds(i * 16, 16)] = jnp.ones((16,), jnp.float32)
        return 0

    lax.fori_loop(0, CHUNK // 16, _fill, 0)

    def _zero(i, _):
        zb_v[pl.ds(i * 16, 16)] = jnp.zeros((16,), jnp.float32)
        return 0

    lax.fori_loop(0, RPT // 16, _zero, 0)
    pltpu.sync_copy(zb_v, acc_sh.at[pl.ds(row0, RPT)])
    plsc.subcore_barrier()

    def _chunk(i, _):
        off = pl.multiple_of(i * CHUNK, CHUNK)
        pltpu.sync_copy(ones_v, acc_sh.at[dst_v.at[pl.ds(off, CHUNK)]], add=True)
        return 0

    lax.fori_loop(0, NCHUNK, _chunk, 0)
    plsc.subcore_barrier()

    @pl.when(cid == 0)
    def _():
        pltpu.sync_copy(acc_sh.at[pl.ds(row0, RPT)], d0_hbm.at[pl.ds(row0, RPT)])

    @pl.when(cid == 1)
    def _():
        pltpu.sync_copy(acc_sh.at[pl.ds(row0, RPT)], d1_hbm.at[pl.ds(row0, RPT)])


@functools.partial(
    pl.kernel,
    mesh=_mesh,
    compiler_params=_sc_params,
    out_type=[
        jax.ShapeDtypeStruct((NP, D), jnp.float32),
        jax.ShapeDtypeStruct((NP, D), jnp.float32),
    ],
    scratch_types=[
        pltpu.VMEM((HALF * C2,), jnp.int32),
        pltpu.VMEM((NCH2 * C2,), jnp.int32),
        pltpu.VMEM((C2,), jnp.int32),
        pltpu.VMEM((C2,), jnp.int32),
        pltpu.VMEM((C2, D), jnp.float32),
        pltpu.VMEM((C2, D), jnp.float32),
        pltpu.VMEM_SHARED((NP, D), jnp.float32),
        pltpu.SemaphoreType.DMA,
        pltpu.SemaphoreType.DMA,
    ],
)
def _sc_aggregate(y_hbm, src_hbm, dst_hbm, p0_hbm, p1_hbm,
                  src_v, dst_v, src_c, dst_c, rows_a, rows_b, acc_sh,
                  sga, sgb):
    cid = lax.axis_index("c")
    sid = lax.axis_index("s")
    wid = cid * NS + sid
    ebase = pl.multiple_of(wid * EPT2, 8)
    row0 = pl.multiple_of(sid * RPT, RPT)

    pltpu.sync_copy(dst_hbm.at[pl.ds(ebase, NCH2 * C2)], dst_v)

    @pl.when(cid == 0)
    def _():
        pltpu.sync_copy(y_hbm.at[pl.ds(row0, RPT)], acc_sh.at[pl.ds(row0, RPT)])

    @pl.when(cid == 1)
    def _():
        def _zero(i, _):
            for j in range(D // 16):
                rows_a[i, pl.ds(j * 16, 16)] = jnp.zeros((16,), jnp.float32)
            return 0

        lax.fori_loop(0, C2, _zero, 0)
        for j in range(RPT // C2):
            pltpu.sync_copy(rows_a, acc_sh.at[pl.ds(row0 + j * C2, C2)])

    plsc.subcore_barrier()

    @pl.when(wid < 4)
    def _():
        lbase = pl.multiple_of(NREG + wid * C2, 8)
        pltpu.sync_copy(src_hbm.at[pl.ds(lbase, C2)], src_c)
        pltpu.sync_copy(dst_hbm.at[pl.ds(lbase, C2)], dst_c)
        pltpu.async_copy(y_hbm.at[src_c], rows_a, sga).wait()
        pltpu.sync_copy(rows_a, acc_sh.at[dst_c], add=True)

    def _start_gather(j, rows, sem):
        off = pl.multiple_of(j * C2, C2)
        pltpu.async_copy(y_hbm.at[src_v.at[pl.ds(off, C2)]], rows, sem)

    def _wait_gather(rows, sem):
        pltpu.make_async_copy(y_hbm.at[src_v.at[pl.ds(0, C2)]], rows, sem).wait()

    def _scatter(i, rows):
        off = pl.multiple_of(i * C2, C2)
        pltpu.sync_copy(rows, acc_sh.at[dst_v.at[pl.ds(off, C2)]], add=True)

    for p in range(2):
        pltpu.sync_copy(
            src_hbm.at[pl.ds(pl.multiple_of(ebase + p * HALF * C2, 8), HALF * C2)],
            src_v,
        )
        pbase = p * HALF

        _start_gather(0, rows_a, sga)
        _start_gather(1, rows_b, sgb)

        def _pair(k, _):
            j = 2 * k
            _wait_gather(rows_a, sga)
            _scatter(pbase + j, rows_a)
            _start_gather(j + 2, rows_a, sga)
            _wait_gather(rows_b, sgb)
            _scatter(pbase + j + 1, rows_b)

            @pl.when(j + 3 < HALF)
            def _():
                _start_gather(j + 3, rows_b, sgb)

            return 0

        lax.fori_loop(0, (HALF - 1) // 2, _pair, 0)
        _wait_gather(rows_a, sga)
        _scatter(pbase + HALF - 1, rows_a)

    plsc.subcore_barrier()

    @pl.when(cid == 0)
    def _():
        pltpu.sync_copy(acc_sh.at[pl.ds(row0, RPT)], p0_hbm.at[pl.ds(row0, RPT)])

    @pl.when(cid == 1)
    def _():
        pltpu.sync_copy(acc_sh.at[pl.ds(row0, RPT)], p1_hbm.at[pl.ds(row0, RPT)])


RB = 400
NG = N_NODES // RB


def _dinv(d0, d1):
    return lax.rsqrt(d0 + d1 + 1.0)


def _y1_body(x_ref, w_ref, d0_ref, d1_ref, o_ref):
    dinv = _dinv(d0_ref[...], d1_ref[...])
    xw = jnp.dot(x_ref[...], w_ref[...], preferred_element_type=jnp.float32)
    o_ref[...] = xw * dinv


def _mid_body(p0_ref, p1_ref, d0_ref, d1_ref, b_ref, w_ref, o_ref):
    dinv = _dinv(d0_ref[...], d1_ref[...])
    h = jnp.maximum(dinv * (p0_ref[...] + p1_ref[...]) + b_ref[...], 0.0)
    o_ref[...] = dinv * jnp.dot(h, w_ref[...], preferred_element_type=jnp.float32)


def _fin_body(p0_ref, p1_ref, d0_ref, d1_ref, b_ref, o_ref):
    dinv = _dinv(d0_ref[...], d1_ref[...])
    o_ref[...] = dinv * (p0_ref[...] + p1_ref[...]) + b_ref[...]


_row_spec = pl.BlockSpec((RB, D), lambda i: (i, 0))
_deg_spec = pl.BlockSpec((RB, 1), lambda i: (i, 0))
_w_spec = pl.BlockSpec((D, D), lambda i: (0, 0))
_b_spec = pl.BlockSpec((1, D), lambda i: (0, 0))
_out_sds = jax.ShapeDtypeStruct((NP, D), jnp.float32)

_tc_y1 = pl.pallas_call(
    _y1_body,
    grid=(NG,),
    in_specs=[_row_spec, _w_spec, _deg_spec, _deg_spec],
    out_specs=_row_spec,
    out_shape=_out_sds,
)

_tc_mid = pl.pallas_call(
    _mid_body,
    grid=(NG,),
    in_specs=[_row_spec, _row_spec, _deg_spec, _deg_spec, _b_spec, _w_spec],
    out_specs=_row_spec,
    out_shape=_out_sds,
)

_tc_fin = pl.pallas_call(
    _fin_body,
    grid=(NG,),
    in_specs=[_row_spec, _row_spec, _deg_spec, _deg_spec, _b_spec],
    out_specs=_row_spec,
    out_shape=jax.ShapeDtypeStruct((N_NODES, D), jnp.float32),
)


@jax.jit
def kernel(x, edge_index, W1, b1, W2, b2):
    src = edge_index[0].astype(jnp.int32)
    dst = edge_index[1].astype(jnp.int32)

    dd0, dd1 = _sc_degree(dst)
    d0 = dd0.reshape(NP, 1)
    d1 = dd1.reshape(NP, 1)
    b1r = b1.reshape(1, D)
    b2r = b2.reshape(1, D)

    y1 = _tc_y1(x, W1, d0, d1)
    p0, p1 = _sc_aggregate(y1, src, dst)
    y2 = _tc_mid(p0, p1, d0, d1, b1r, W2)
    q0, q1 = _sc_aggregate(y2, src, dst)
    return _tc_fin(q0, q1, d0, d1, b2r)
